# baseline (speedup 1.0000x reference)
import functools

import jax
import jax.numpy as jnp
from jax import lax
from jax.experimental import pallas as pl
from jax.experimental.pallas import tpu as pltpu
from jax.experimental.pallas import tpu_sc as plsc

NC = 2
NS = 16
L = 16
NW = NC * NS
BATCH = 16384
D = 16
BPW = BATCH // NW
GCHUNK = 128
NG = BPW // GCHUNK
CHUNKS = BPW // L


def _gmf_body(uidx_hbm, iidx_hbm, utab_hbm, itab_hbm, wmat_hbm, out_hbm,
              uidx_v, iidx_v, urows_v, irows_v, wmat_v, out_v, sem_u, sem_i):
    wid = lax.axis_index("s") * NC + lax.axis_index("c")
    base = wid * BPW

    pltpu.sync_copy(uidx_hbm.at[pl.ds(base, BPW)], uidx_v)
    pltpu.sync_copy(iidx_hbm.at[pl.ds(base, BPW)], iidx_v)
    pltpu.sync_copy(wmat_hbm, wmat_v)

    copies = []
    for j in range(NG):
        sl = pl.ds(j * GCHUNK, GCHUNK)
        copies.append(pltpu.async_copy(
            utab_hbm.at[uidx_v.at[sl]], urows_v.at[sl], sem_u))
        copies.append(pltpu.async_copy(
            itab_hbm.at[iidx_v.at[sl]], irows_v.at[sl], sem_i))
    for c in copies:
        c.wait()

    bias = wmat_v[D, :]

    def chunk_body(c, carry):
        rows = c * L + lax.broadcasted_iota(jnp.int32, (L,), 0)
        acc = bias
        for k in range(D):
            col = jnp.full((L,), k, jnp.int32)
            uk = plsc.load_gather(urows_v, [rows, col])
            tk = plsc.load_gather(irows_v, [rows, col])
            acc = acc + uk * tk * wmat_v[k, :]
        out_v[pl.ds(c * L, L)] = 1.0 / (1.0 + jnp.exp(-acc))
        return carry

    lax.fori_loop(0, CHUNKS, chunk_body, 0)

    pltpu.sync_copy(out_v, out_hbm.at[pl.ds(base, BPW)])


@functools.partial(jax.jit, static_argnames=())
def kernel(user_idxs, item_idxs, user_table, item_table, W, b):
    uidx = user_idxs.astype(jnp.int32)
    iidx = item_idxs.astype(jnp.int32)
    wmat = jnp.concatenate(
        [jnp.broadcast_to(W, (D, L)),
         jnp.broadcast_to(b.reshape(1, 1), (1, L))], axis=0)

    mesh = plsc.VectorSubcoreMesh(core_axis_name="c", subcore_axis_name="s")
    k = pl.kernel(
        _gmf_body,
        out_type=jax.ShapeDtypeStruct((BATCH,), jnp.float32),
        mesh=mesh,
        compiler_params=pltpu.CompilerParams(
            needs_layout_passes=False, use_tc_tiling_on_sc=False),
        scratch_types=[
            pltpu.VMEM((BPW,), jnp.int32),
            pltpu.VMEM((BPW,), jnp.int32),
            pltpu.VMEM((BPW, D), jnp.float32),
            pltpu.VMEM((BPW, D), jnp.float32),
            pltpu.VMEM((D + 1, L), jnp.float32),
            pltpu.VMEM((BPW,), jnp.float32),
            pltpu.SemaphoreType.DMA,
            pltpu.SemaphoreType.DMA,
        ],
    )
    return k(uidx, iidx, user_table, item_table, wmat)


# speedup vs baseline: 7.0527x; 7.0527x over previous
import functools

import jax
import jax.numpy as jnp
from jax import lax
from jax.experimental import pallas as pl
from jax.experimental.pallas import tpu as pltpu
from jax.experimental.pallas import tpu_sc as plsc

NC = 2
NS = 16
L = 16
NW = NC * NS
BATCH = 16384
D = 16
BPW = BATCH // NW
W128 = 128
NCH = BPW // L


def _gmf_body(uidx_hbm, iidx_hbm, utab_hbm, itab_hbm, wmat_hbm, out_hbm,
              uidx_v, iidx_v, stag, urows, irows, wmat_v,
              out_v, sem):
    wid = lax.axis_index("s") * NC + lax.axis_index("c")
    base = wid * BPW

    pltpu.sync_copy(uidx_hbm.at[pl.ds(base, BPW)], uidx_v)
    pltpu.sync_copy(iidx_hbm.at[pl.ds(base, BPW)], iidx_v)
    pltpu.sync_copy(wmat_hbm, wmat_v)

    lane = lax.broadcasted_iota(jnp.int32, (L,), 0)

    def one_table(tab_hbm, idx_v, rows):
        def fetch(c, sl):
            idxv = idx_v[pl.ds(c * L, L)]
            for e in range(L):
                r = idxv[e]
                rb = pl.multiple_of(r & ~(W128 - 1), W128)
                pltpu.async_copy(tab_hbm.at[:, pl.ds(rb, W128)],
                                 stag.at[sl, e], sem)

        def drain():
            def wt(e, carry):
                pltpu.make_async_copy(tab_hbm.at[:, pl.ds(0, W128)],
                                      stag.at[0, 0], sem).wait()
                return carry
            lax.fori_loop(0, L, wt, 0)

        def extract(c, sl):
            rpos = idx_v[pl.ds(c * L, L)] & (W128 - 1)
            slv = jnp.full((L,), sl, jnp.int32)
            rid = jnp.full((L,), c * L, jnp.int32) + lane
            for k in range(D):
                kv = jnp.full((L,), k, jnp.int32)
                v = plsc.load_gather(stag, [slv, lane, kv, rpos])
                plsc.store_scatter(rows, [rid * D + kv], v)

        fetch(0, 0)

        def step(c, carry):
            sl = c % 2

            @pl.when(c + 1 < NCH)
            def _():
                fetch(c + 1, 1 - sl)

            drain()
            extract(c, sl)
            return carry

        lax.fori_loop(0, NCH, step, 0)

    one_table(utab_hbm, uidx_v, urows)
    one_table(itab_hbm, iidx_v, irows)

    bias = wmat_v[D, :]

    def combine(g, carry):
        rid = g * L + lane
        acc = bias
        for k in range(D):
            kv = jnp.full((L,), k, jnp.int32)
            vu = plsc.load_gather(urows, [rid * D + kv])
            vi = plsc.load_gather(irows, [rid * D + kv])
            acc = acc + vu * vi * wmat_v[k, :]
        out_v[pl.ds(g * L, L)] = 1.0 / (1.0 + jnp.exp(-acc))
        return carry

    lax.fori_loop(0, NCH, combine, 0)

    pltpu.sync_copy(out_v, out_hbm.at[pl.ds(base, BPW)])


@functools.partial(jax.jit, static_argnames=())
def kernel(user_idxs, item_idxs, user_table, item_table, W, b):
    uidx = user_idxs.astype(jnp.int32)
    iidx = item_idxs.astype(jnp.int32)
    utab_t = user_table.T
    itab_t = item_table.T
    wmat = jnp.concatenate(
        [jnp.broadcast_to(W, (D, L)),
         jnp.broadcast_to(b.reshape(1, 1), (1, L))], axis=0)

    mesh = plsc.VectorSubcoreMesh(core_axis_name="c", subcore_axis_name="s")
    k = pl.kernel(
        _gmf_body,
        out_type=jax.ShapeDtypeStruct((BATCH,), jnp.float32),
        mesh=mesh,
        compiler_params=pltpu.CompilerParams(
            needs_layout_passes=False, use_tc_tiling_on_sc=True),
        scratch_types=[
            pltpu.VMEM((BPW,), jnp.int32),
            pltpu.VMEM((BPW,), jnp.int32),
            pltpu.VMEM((2, L, D, W128), jnp.float32),
            pltpu.VMEM((BPW * D,), jnp.float32),
            pltpu.VMEM((BPW * D,), jnp.float32),
            pltpu.VMEM((D + 1, L), jnp.float32),
            pltpu.VMEM((BPW,), jnp.float32),
            pltpu.SemaphoreType.DMA,
        ],
    )
    return k(uidx, iidx, utab_t, itab_t, wmat)
